# baseline (speedup 1.0000x reference)
import jax
import jax.numpy as jnp
from jax.experimental import pallas as pl
from jax.experimental.pallas import tpu as pltpu
from jax.experimental.pallas import tpu_sc as plsc

B = 4096
L = 50
D = 128
TIME_NUM = 50
TIME_PAD = 64
MAX_TIME = 1000.0
GATHER_WINDOW = 128
NC = 8
BC = B // NC


def _sc_gather(table, flat_idx):
    n = flat_idx.shape[0]
    idx2d = flat_idx.reshape(1, n)
    mesh = plsc.VectorSubcoreMesh(core_axis_name="core", subcore_axis_name="subcore")

    @pl.kernel(
        out_type=jax.ShapeDtypeStruct((n, table.shape[1]), table.dtype),
        mesh=mesh,
    )
    def kern(x_hbm, i_hbm, o_hbm):
        def body(i_vmem, o_vmem):
            pltpu.sync_copy(x_hbm.at[i_vmem.at[0]], o_vmem)

        pltpu.emit_pipeline(
            body,
            grid=(n // GATHER_WINDOW,),
            in_specs=[pl.BlockSpec((1, GATHER_WINDOW), index_map=lambda i: (0, i))],
            out_specs=[
                pl.BlockSpec((GATHER_WINDOW, table.shape[1]), index_map=lambda i: (i, 0))
            ],
            core_axis_name=("core", "subcore"),
            dimension_semantics=(pltpu.PARALLEL,),
        )(i_hbm, o_hbm)

    return kern(table, idx2d)


def _lstm_kernel(maxlen_ref, x_ref, tidx_ref, len_ref, pos_ref, time_ref,
                 wih_ref, whh_ref, bias_ref, wtr_ref, btr_ref, out_ref,
                 h_ref, c_ref):
    c_id = pl.program_id(0)
    t = pl.program_id(1)
    m = maxlen_ref[c_id]

    @pl.when(t == 0)
    def _():
        h_ref[...] = jnp.zeros_like(h_ref)
        c_ref[...] = jnp.zeros_like(c_ref)

    @pl.when(t < m)
    def _():
        xt = x_ref[0]
        tcol = tidx_ref[0]
        onehot = (tcol == jax.lax.broadcasted_iota(
            jnp.int32, (BC, TIME_PAD), 1)).astype(jnp.bfloat16)
        xt = xt + jnp.dot(onehot, time_ref[...],
                          preferred_element_type=jnp.float32)
        xt = xt + pos_ref[0]

        h = h_ref[...]
        c = c_ref[...]
        gates = (jnp.dot(xt.astype(jnp.bfloat16), wih_ref[...],
                         preferred_element_type=jnp.float32)
                 + jnp.dot(h.astype(jnp.bfloat16), whh_ref[...],
                           preferred_element_type=jnp.float32)
                 + bias_ref[...])
        gi = jax.nn.sigmoid(gates[:, 0:D])
        gf = jax.nn.sigmoid(gates[:, D:2 * D])
        gg = jnp.tanh(gates[:, 2 * D:3 * D])
        go = jax.nn.sigmoid(gates[:, 3 * D:4 * D])
        c_new = gf * c + gi * gg
        h_new = go * jnp.tanh(c_new)
        mask = t < len_ref[...]
        h = jnp.where(mask, h_new, h)
        h_ref[...] = h
        c_ref[...] = jnp.where(mask, c_new, c)

        @pl.when(t == m - 1)
        def _():
            out_ref[...] = jax.nn.relu(
                jnp.dot(h.astype(jnp.bfloat16), wtr_ref[...],
                        preferred_element_type=jnp.float32)
                + btr_ref[...])


def _run_lstm(maxlen, x_lbd, tidx_t, len2d, pos_slice, time_pad, wih_t, whh_t,
              bias, wtr_t, btr):
    def xmap(c, t, m):
        return (jnp.minimum(t, m[c] - 1), c, 0)

    def posmap(c, t, m):
        return (jnp.minimum(t, m[c] - 1), 0, 0)

    grid_spec = pltpu.PrefetchScalarGridSpec(
        num_scalar_prefetch=1,
        grid=(NC, L),
        in_specs=[
            pl.BlockSpec((1, BC, D), xmap),
            pl.BlockSpec((1, BC, 1), xmap),
            pl.BlockSpec((BC, 1), lambda c, t, m: (c, 0)),
            pl.BlockSpec((1, 1, D), posmap),
            pl.BlockSpec((TIME_PAD, D), lambda c, t, m: (0, 0)),
            pl.BlockSpec((D, 4 * D), lambda c, t, m: (0, 0)),
            pl.BlockSpec((D, 4 * D), lambda c, t, m: (0, 0)),
            pl.BlockSpec((1, 4 * D), lambda c, t, m: (0, 0)),
            pl.BlockSpec((D, D), lambda c, t, m: (0, 0)),
            pl.BlockSpec((1, D), lambda c, t, m: (0, 0)),
        ],
        out_specs=pl.BlockSpec((BC, D), lambda c, t, m: (c, 0)),
        scratch_shapes=[
            pltpu.VMEM((BC, D), jnp.float32),
            pltpu.VMEM((BC, D), jnp.float32),
        ],
    )
    return pl.pallas_call(
        _lstm_kernel,
        grid_spec=grid_spec,
        out_shape=jax.ShapeDtypeStruct((B, D), jnp.float32),
        compiler_params=pltpu.CompilerParams(
            dimension_semantics=("arbitrary", "arbitrary")),
    )(maxlen, x_lbd, tidx_t, len2d, pos_slice, time_pad, wih_t, whh_t, bias,
      wtr_t, btr)


def kernel(static_table, time_table, pos_table, W_ih, W_hh, b_ih, b_hh,
           W_trans, b_trans, cas_times, cas_history, lengths):
    perm = jnp.argsort(lengths)
    inv = jnp.zeros((B,), jnp.int32).at[perm].set(
        jnp.arange(B, dtype=jnp.int32))
    lengths_s = lengths[perm]
    maxlen = jnp.max(lengths_s.reshape(NC, BC), axis=1)

    tidx = jnp.clip(
        jnp.floor(cas_times / MAX_TIME * TIME_NUM).astype(jnp.int32),
        0, TIME_NUM - 1)
    tidx_t = tidx[perm].T.reshape(L, B, 1)
    idx_flat = cas_history[perm].T.reshape(L * B)
    x_lbd = _sc_gather(static_table, idx_flat).reshape(L, B, D)

    pos_slice = pos_table[:L].reshape(L, 1, D)
    time_pad = jnp.zeros((TIME_PAD, D), jnp.float32).at[:TIME_NUM].set(
        time_table).astype(jnp.bfloat16)
    bias = (b_ih + b_hh).reshape(1, 4 * D)
    out_s = _run_lstm(maxlen, x_lbd, tidx_t, lengths_s.reshape(B, 1),
                      pos_slice, time_pad,
                      W_ih.T.astype(jnp.bfloat16), W_hh.T.astype(jnp.bfloat16),
                      bias, W_trans.T.astype(jnp.bfloat16),
                      b_trans.reshape(1, D))
    return jnp.take(out_s, inv, axis=0)
